# baseline (speedup 1.0000x reference)
import jax
import jax.numpy as jnp
from jax.experimental import pallas as pl
from jax.experimental.pallas import tpu as pltpu

B, T, C = 4, 2048, 768
T_OUT = 2048
BC = 256


def _body(w_ref, x_ref, b_ref, o_ref, wt_ref):
    b_id = pl.program_id(0)
    c_id = pl.program_id(1)

    @pl.when(jnp.logical_and(b_id == 0, c_id == 0))
    def _():
        wt_ref[...] = w_ref[...].T.astype(jnp.bfloat16)

    x = x_ref[0]
    bias = b_ref[...]

    acc = jax.lax.dot_general(
        wt_ref[...], x.astype(jnp.bfloat16), (((1,), (0,)), ((), ())),
        preferred_element_type=jnp.float32,
    )
    d = jnp.tanh(acc + bias)

    xm1 = jnp.concatenate([x[:1], x[:-1]], axis=0)
    xp1 = jnp.concatenate([x[1:], x[-1:]], axis=0)
    out = (x + jnp.maximum(d, 0.0) * (xp1 - x)
             + jnp.minimum(d, 0.0) * (x - xm1))

    zero = jnp.zeros((1, BC), jnp.float32)
    r0 = jnp.where(d[:1] < 0.0, zero, out[:1])
    rl = jnp.where(d[-1:] < 0.0, out[-1:], zero)
    rp = jnp.where(d[T - 2:T - 1] >= 1.0, zero, out[T - 2:T - 1])
    o_ref[0] = jnp.concatenate([r0, out[1:T - 2], rp, rl], axis=0)


@jax.jit
def kernel(X, W_loc, b_loc):
    bias = b_loc.reshape(T_OUT, 1)
    grid = (B, C // BC)
    return pl.pallas_call(
        _body,
        grid=grid,
        in_specs=[
            pl.BlockSpec((T, T_OUT), lambda b, c: (0, 0)),
            pl.BlockSpec((1, T, BC), lambda b, c: (b, 0, c)),
            pl.BlockSpec((T_OUT, 1), lambda b, c: (0, 0)),
        ],
        out_specs=pl.BlockSpec((1, T_OUT, BC), lambda b, c: (b, 0, c)),
        out_shape=jax.ShapeDtypeStruct((B, T_OUT, C), jnp.float32),
        scratch_shapes=[pltpu.VMEM((T_OUT, T), jnp.bfloat16)],
        compiler_params=pltpu.CompilerParams(
            dimension_semantics=("arbitrary", "arbitrary"),
        ),
    )(W_loc, X, bias)


# speedup vs baseline: 1.0106x; 1.0106x over previous
import jax
import jax.numpy as jnp
from jax.experimental import pallas as pl
from jax.experimental.pallas import tpu as pltpu

B, T, C = 4, 2048, 768
T_OUT = 2048
BC = 256


def _body(w_ref, x_ref, b_ref, o_ref, wt_ref):
    b_id = pl.program_id(0)
    c_id = pl.program_id(1)

    @pl.when(jnp.logical_and(b_id == 0, c_id == 0))
    def _():
        wt_ref[...] = w_ref[...].T.astype(jnp.bfloat16)

    x = x_ref[0]
    bias = b_ref[...]

    acc = jax.lax.dot_general(
        wt_ref[...], x.astype(jnp.bfloat16), (((1,), (0,)), ((), ())),
        preferred_element_type=jnp.float32,
    )
    d = jnp.tanh(acc + bias)

    neg = d < 0.0
    sat = d >= 1.0
    w1 = d - jnp.floor(d)
    xm1 = jnp.concatenate([x[:1], x[:-1]], axis=0)
    xp1 = jnp.concatenate([x[1:], x[-1:]], axis=0)
    va = jnp.where(neg, xm1, jnp.where(sat, xp1, x))
    vb = jnp.where(neg, x, xp1)
    out = va + w1 * (vb - va)

    zero = jnp.zeros((1, BC), jnp.float32)
    r0 = jnp.where(neg[:1], zero, out[:1])
    rl = jnp.where(neg[-1:], out[-1:], zero)
    rp = jnp.where(sat[T - 2:T - 1], zero, out[T - 2:T - 1])
    o_ref[0] = jnp.concatenate([r0, out[1:T - 2], rp, rl], axis=0)


@jax.jit
def kernel(X, W_loc, b_loc):
    bias = b_loc.reshape(T_OUT, 1)
    grid = (B, C // BC)
    return pl.pallas_call(
        _body,
        grid=grid,
        in_specs=[
            pl.BlockSpec((T, T_OUT), lambda b, c: (0, 0)),
            pl.BlockSpec((1, T, BC), lambda b, c: (b, 0, c)),
            pl.BlockSpec((T_OUT, 1), lambda b, c: (0, 0)),
        ],
        out_specs=pl.BlockSpec((1, T_OUT, BC), lambda b, c: (b, 0, c)),
        out_shape=jax.ShapeDtypeStruct((B, T_OUT, C), jnp.float32),
        scratch_shapes=[pltpu.VMEM((T_OUT, T), jnp.bfloat16)],
        compiler_params=pltpu.CompilerParams(
            dimension_semantics=("arbitrary", "arbitrary"),
        ),
    )(W_loc, X, bias)
